# baseline (speedup 1.0000x reference)
import functools

import jax
import jax.numpy as jnp
from jax import lax
from jax.experimental import pallas as pl
from jax.experimental.pallas import tpu as pltpu
from jax.experimental.pallas import tpu_sc as plsc

N = 524288
V = 100000
F = 16
NC, NS = 2, 16
NW = NC * NS
L = 16

C1 = 800
NCH1 = V // C1
PTS_W = N // NW
C2 = 1024

_mesh = plsc.VectorSubcoreMesh(core_axis_name="c", subcore_axis_name="s")
_params = pltpu.CompilerParams(use_tc_tiling_on_sc=False,
                               needs_layout_passes=False)


@functools.partial(
    pl.kernel,
    mesh=_mesh,
    compiler_params=_params,
    out_type=jax.ShapeDtypeStruct((V, F), jnp.float32),
    scratch_types=[
        pltpu.VMEM((C1,), jnp.int32),
        pltpu.VMEM((C1,), jnp.int32),
        pltpu.VMEM((C1,), jnp.int32),
        pltpu.VMEM((C1, F), jnp.float32),
        pltpu.VMEM((C1, F), jnp.float32),
        pltpu.VMEM((C1, F), jnp.float32),
        pltpu.SemaphoreType.DMA,
        pltpu.SemaphoreType.DMA,
        pltpu.SemaphoreType.DMA,
    ],
)
def _fuse(m0h, m1h, m2h, f0h, f1h, f2h, outh,
          m0, m1, m2, r0, r1, r2, s0, s1, s2):
    wid = lax.axis_index("s") * NC + lax.axis_index("c")
    nch = (NCH1 - wid + NW - 1) // NW

    def chunk(ci, carry):
        off = (wid + ci * NW) * C1
        pltpu.sync_copy(m0h.at[pl.ds(off, C1)], m0)
        pltpu.sync_copy(m1h.at[pl.ds(off, C1)], m1)
        pltpu.sync_copy(m2h.at[pl.ds(off, C1)], m2)
        cp0 = pltpu.async_copy(f0h.at[m0], r0, s0)
        cp1 = pltpu.async_copy(f1h.at[m1], r1, s1)
        cp2 = pltpu.async_copy(f2h.at[m2], r2, s2)
        cp0.wait()
        cp1.wait()
        cp2.wait()

        def add_row(i, c):
            r0[i, :] = r0[i, :] + r1[i, :] + r2[i, :]
            return c

        lax.fori_loop(0, C1, add_row, 0, unroll=8)
        pltpu.sync_copy(r0, outh.at[pl.ds(off, C1)])
        return carry

    lax.fori_loop(0, nch, chunk, 0)


@functools.partial(
    pl.kernel,
    mesh=_mesh,
    compiler_params=_params,
    out_type=jax.ShapeDtypeStruct((N, F), jnp.float32),
    scratch_types=[
        pltpu.VMEM((3 * C2,), jnp.int32),
        pltpu.VMEM((3 * C2,), jnp.float32),
        pltpu.VMEM((3 * C2, F), jnp.float32),
        pltpu.VMEM((C2, F), jnp.float32),
        pltpu.SemaphoreType.DMA,
    ],
)
def _interp(th, bh, tabh, outh, ti, bi, r, o, s0):
    wid = lax.axis_index("s") * NC + lax.axis_index("c")
    base = wid * PTS_W
    lanes = lax.iota(jnp.int32, L)

    def chunk(ci, carry):
        off = base + ci * C2
        pltpu.sync_copy(th.at[pl.ds(3 * off, 3 * C2)], ti)
        cp = pltpu.async_copy(tabh.at[ti], r, s0)
        pltpu.sync_copy(bh.at[pl.ds(3 * off, 3 * C2)], bi)
        cp.wait()

        def group(g, c):
            gbase = g * L
            flat = (gbase + lanes) * 3
            bv0 = plsc.load_gather(bi, [flat])
            bv1 = plsc.load_gather(bi, [flat + 1])
            bv2 = plsc.load_gather(bi, [flat + 2])
            for p in range(L):
                q = gbase + p
                o[q, :] = (bv0[p] * r[3 * q, :] + bv1[p] * r[3 * q + 1, :]
                           + bv2[p] * r[3 * q + 2, :])
            return c

        lax.fori_loop(0, C2 // L, group, 0)
        pltpu.sync_copy(o, outh.at[pl.ds(off, C2)])
        return carry

    lax.fori_loop(0, PTS_W // C2, chunk, 0)


def kernel(bary, triangle, feat0, feat1, feat2, map0, map1, map2):
    tab = _fuse(map0, map1, map2, feat0, feat1, feat2)
    return _interp(triangle.reshape(-1), bary.reshape(-1), tab)


# speedup vs baseline: 3.6120x; 3.6120x over previous
import functools

import jax
import jax.numpy as jnp
from jax import lax
from jax.experimental import pallas as pl
from jax.experimental.pallas import tpu as pltpu
from jax.experimental.pallas import tpu_sc as plsc

N = 524288
V = 100000
F = 16
NC, NS = 2, 16
NW = NC * NS
L = 16

C1 = 800
NCH1 = V // C1
PTS_W = N // NW
C2 = 1024

_mesh = plsc.VectorSubcoreMesh(core_axis_name="c", subcore_axis_name="s")
_params = pltpu.CompilerParams(use_tc_tiling_on_sc=False)


@functools.partial(
    pl.kernel,
    mesh=_mesh,
    compiler_params=_params,
    out_type=jax.ShapeDtypeStruct((V, F), jnp.float32),
    scratch_types=[
        pltpu.VMEM((C1,), jnp.int32),
        pltpu.VMEM((C1,), jnp.int32),
        pltpu.VMEM((C1,), jnp.int32),
        pltpu.VMEM((C1, F), jnp.float32),
        pltpu.VMEM((C1, F), jnp.float32),
        pltpu.VMEM((C1, F), jnp.float32),
        pltpu.SemaphoreType.DMA,
        pltpu.SemaphoreType.DMA,
        pltpu.SemaphoreType.DMA,
    ],
)
def _fuse(m0h, m1h, m2h, f0h, f1h, f2h, outh,
          m0, m1, m2, r0, r1, r2, s0, s1, s2):
    wid = lax.axis_index("s") * NC + lax.axis_index("c")
    nch = (NCH1 - wid + NW - 1) // NW

    def chunk(ci, carry):
        off = (wid + ci * NW) * C1
        pltpu.sync_copy(m0h.at[pl.ds(off, C1)], m0)
        pltpu.sync_copy(m1h.at[pl.ds(off, C1)], m1)
        pltpu.sync_copy(m2h.at[pl.ds(off, C1)], m2)
        cp0 = pltpu.async_copy(f0h.at[m0], r0, s0)
        cp1 = pltpu.async_copy(f1h.at[m1], r1, s1)
        cp2 = pltpu.async_copy(f2h.at[m2], r2, s2)
        cp0.wait()
        cp1.wait()
        cp2.wait()

        def add_row(i, c):
            r0[i, :] = r0[i, :] + r1[i, :] + r2[i, :]
            return c

        lax.fori_loop(0, C1, add_row, 0, unroll=8)
        pltpu.sync_copy(r0, outh.at[pl.ds(off, C1)])
        return carry

    lax.fori_loop(0, nch, chunk, 0)


@functools.partial(
    pl.kernel,
    mesh=_mesh,
    compiler_params=_params,
    out_type=jax.ShapeDtypeStruct((N, F), jnp.float32),
    scratch_types=[
        pltpu.VMEM((C2,), jnp.int32),
        pltpu.VMEM((C2,), jnp.int32),
        pltpu.VMEM((C2,), jnp.int32),
        pltpu.VMEM((C2,), jnp.float32),
        pltpu.VMEM((C2,), jnp.float32),
        pltpu.VMEM((C2,), jnp.float32),
        pltpu.VMEM((C2, F), jnp.float32),
        pltpu.VMEM((C2, F), jnp.float32),
        pltpu.VMEM((C2, F), jnp.float32),
        pltpu.VMEM((C2, F), jnp.float32),
        pltpu.SemaphoreType.DMA,
        pltpu.SemaphoreType.DMA,
        pltpu.SemaphoreType.DMA,
    ],
)
def _interp(t0h, t1h, t2h, b0h, b1h, b2h, tabh, outh,
            i0, i1, i2, b0, b1, b2, r0, r1, r2, o, s0, s1, s2):
    wid = lax.axis_index("s") * NC + lax.axis_index("c")
    base = wid * PTS_W

    def chunk(ci, carry):
        off = base + ci * C2
        pltpu.sync_copy(t0h.at[pl.ds(off, C2)], i0)
        pltpu.sync_copy(t1h.at[pl.ds(off, C2)], i1)
        pltpu.sync_copy(t2h.at[pl.ds(off, C2)], i2)
        cp0 = pltpu.async_copy(tabh.at[i0], r0, s0)
        cp1 = pltpu.async_copy(tabh.at[i1], r1, s1)
        cp2 = pltpu.async_copy(tabh.at[i2], r2, s2)
        pltpu.sync_copy(b0h.at[pl.ds(off, C2)], b0)
        pltpu.sync_copy(b1h.at[pl.ds(off, C2)], b1)
        pltpu.sync_copy(b2h.at[pl.ds(off, C2)], b2)
        cp0.wait()
        cp1.wait()
        cp2.wait()

        def group(g, c):
            gbase = g * L
            bv0 = b0[pl.ds(gbase, L)]
            bv1 = b1[pl.ds(gbase, L)]
            bv2 = b2[pl.ds(gbase, L)]
            for p in range(L):
                q = gbase + p
                o[q, :] = (bv0[p] * r0[q, :] + bv1[p] * r1[q, :]
                           + bv2[p] * r2[q, :])
            return c

        lax.fori_loop(0, C2 // L, group, 0)
        pltpu.sync_copy(o, outh.at[pl.ds(off, C2)])
        return carry

    lax.fori_loop(0, PTS_W // C2, chunk, 0)


def kernel(bary, triangle, feat0, feat1, feat2, map0, map1, map2):
    tab = _fuse(map0, map1, map2, feat0, feat1, feat2)
    tri_t = triangle.T
    bary_t = bary.T
    return _interp(tri_t[0], tri_t[1], tri_t[2],
                   bary_t[0], bary_t[1], bary_t[2], tab)
